# baseline (speedup 1.0000x reference)
import functools

import jax
import jax.numpy as jnp
from jax import lax
from jax.experimental import pallas as pl
from jax.experimental.pallas import tpu as pltpu
from jax.experimental.pallas import tpu_sc as plsc

B, T, N, D = 16, 2048, 8, 128
TOK = B * T
ROW = N * D
NC, NS, L = 2, 16, 16
NW = NC * NS
TPW = TOK // NW
CH = 32
NCH = TPW // CH

_mesh = plsc.VectorSubcoreMesh(core_axis_name="c", subcore_axis_name="s")


@functools.partial(
    pl.kernel,
    out_type=jax.ShapeDtypeStruct((TOK, ROW), jnp.float32),
    mesh=_mesh,
    scratch_types=[
        pltpu.VMEM((ROW,), jnp.float32),
        pltpu.VMEM((TPW * N,), jnp.int32),
        pltpu.VMEM((2, CH, ROW), jnp.float32),
        pltpu.SemaphoreType.DMA,
        pltpu.SemaphoreType.DMA,
    ],
)
def _sc_embed(bp_hbm, w_hbm, out_hbm, w_v, bp_v, ob, sem0, sem1):
    wid = lax.axis_index("s") * NC + lax.axis_index("c")
    base = wid * TPW
    pltpu.sync_copy(w_hbm, w_v)
    pltpu.sync_copy(bp_hbm.at[pl.ds(base * N, TPW * N)], bp_v)
    sems = (sem0, sem1)

    def compute_chunk(g, slot):
        def tok_body(t, carry):
            off = (g * CH + 2 * t) * N
            bpv = bp_v[pl.ds(off, 2 * N)].astype(jnp.float32)
            for tt in range(2):
                for n in range(N):
                    s = bpv[tt * N + n]
                    for c in range(D // L):
                        w_vec = w_v[pl.ds(n * D + c * L, L)]
                        ob[slot, 2 * t + tt, pl.ds(n * D + c * L, L)] = w_vec * s
            return carry

        lax.fori_loop(0, CH // 2, tok_body, 0)

    def outer(gp, carry):
        for b in range(2):
            g = gp * 2 + b

            @pl.when(gp > 0)
            def _wait():
                pltpu.make_async_copy(
                    ob.at[b], out_hbm.at[pl.ds(0, CH)], sems[b]
                ).wait()

            compute_chunk(g, b)
            pltpu.async_copy(
                ob.at[b], out_hbm.at[pl.ds(base + g * CH, CH)], sems[b]
            )
        return carry

    lax.fori_loop(0, NCH // 2, outer, 0)
    for b in range(2):
        pltpu.make_async_copy(ob.at[b], out_hbm.at[pl.ds(0, CH)], sems[b]).wait()


def kernel(button_presses, W):
    bp = button_presses.reshape(TOK * N)
    out = _sc_embed(bp, W.reshape(ROW))
    return out.reshape(B, T, N, D)
